# baseline (speedup 1.0000x reference)
import functools

import jax
import jax.numpy as jnp
from jax import lax
from jax.experimental import pallas as pl
from jax.experimental.pallas import tpu as pltpu
from jax.experimental.pallas import tpu_sc as plsc

_N = 2048
_D = 64
_KP1 = 17
_BLK = 256
_KEY_MASK = ~2047
_BIG = 0x7FFFFFFF

_NC = 2
_NS = 16
_NW = _NC * _NS
_NCH = 1
_CN = _N // _NCH
_RPW = _CN // _NW


def _tc_pack_body(chunk, pos_ref, out_ref):
    i = pl.program_id(0)
    p = pos_ref[...]
    a = pos_ref[pl.ds(chunk * _CN + i * _BLK, _BLK), :]
    g = lax.dot_general(a, p, (((1,), (1,)), ((), ())),
                        preferred_element_type=jnp.float32)
    na = jnp.sum(a * a, axis=1, keepdims=True)
    nb = jnp.sum(p * p, axis=1)[None, :]
    s = jnp.maximum(na + nb - 2.0 * g, 0.0)
    col = lax.broadcasted_iota(jnp.int32, s.shape, 1)
    row = lax.broadcasted_iota(jnp.int32, s.shape, 0) + chunk * _CN + i * _BLK
    s = jnp.where(col == row, 0.0, s)
    nrm = jnp.sqrt(s)
    bits = lax.bitcast_convert_type(nrm, jnp.int32)
    out_ref[...] = (bits & jnp.int32(_KEY_MASK)) | col


def _tc_pack(positions, chunk):
    return pl.pallas_call(
        functools.partial(_tc_pack_body, chunk),
        grid=(_CN // _BLK,),
        in_specs=[
            pl.BlockSpec((_N, _D), lambda i: (0, 0)),
        ],
        out_specs=pl.BlockSpec((_BLK, _N), lambda i: (i, 0)),
        out_shape=jax.ShapeDtypeStruct((_CN, _N), jnp.int32),
    )(positions)


def _tree_min(vs):
    while len(vs) > 1:
        nxt = [jnp.minimum(vs[i], vs[i + 1]) for i in range(0, len(vs) - 1, 2)]
        if len(vs) % 2:
            nxt.append(vs[-1])
        vs = nxt
    return vs[0]


def _merge2(ra, da, rb, db):
    cd = lax.rev(rb, (0,))
    m = jnp.minimum(ra, cd)
    x = jnp.maximum(ra, cd)
    return jnp.sort(m), jnp.minimum(jnp.minimum(da, db), jnp.min(x))


def _merge_tree(sorted_chunks):
    big = jnp.int32(_BIG)
    lvl = [(c, big) for c in sorted_chunks]
    while len(lvl) > 1:
        nxt = [_merge2(*lvl[i], *lvl[i + 1]) for i in range(0, len(lvl) - 1, 2)]
        if len(lvl) % 2:
            nxt.append(lvl[-1])
        lvl = nxt
    return lvl[0]


def _sc_topk_body(keys_hbm, out_hbm, buf0, buf1, ob, sem0, sem1):
    wid = lax.axis_index("s") * _NC + lax.axis_index("c")
    base_row = wid * _RPW
    gather_off = lax.iota(jnp.int32, 16) * 16
    big16 = jnp.full((16,), _BIG, jnp.int32)

    def row_topk(buf, row, slot):
        groups = []
        for g in range(8):
            chunks = [buf[row, pl.ds(g * 256 + i * 16, 16)] for i in range(16)]
            groups.append(_tree_min(chunks))
        r, d = _merge_tree([jnp.sort(g) for g in groups])
        row16 = jnp.broadcast_to(row, (16,))
        def block_chunk(key_scalar):
            c = key_scalar & 2047
            base = lax.shift_right_logical(c, 8) * 256 + (c & 15)
            return plsc.load_gather(buf, [row16, gather_off + base])

        cands = [jnp.sort(block_chunk(r[j])) for j in range(16)]
        cands.append(jnp.sort(block_chunk(d)))
        r2, d2 = _merge_tree(cands)
        km = jnp.int32(_KEY_MASK)
        ob[pl.ds(slot * _KP1 + 16, 16)] = plsc.bitcast(
            jnp.broadcast_to(d2 & km, (16,)), jnp.float32)
        ob[pl.ds(slot * _KP1, 16)] = plsc.bitcast(r2 & km, jnp.float32)

    rb = 8

    def wait_batch(sem):
        pltpu.make_async_copy(keys_hbm.at[pl.ds(0, rb)], buf0, sem).wait()

    pltpu.async_copy(keys_hbm.at[pl.ds(base_row, rb)], buf0, sem0)

    def outer(i, carry):
        r0 = base_row + i * (2 * rb)

        iv = 4

        def inner0(j, c):
            for t in range(iv):
                row_topk(buf0, iv * j + t, i * (2 * rb) + iv * j + t)
            return c

        def inner1(j, c):
            for t in range(iv):
                row_topk(buf1, iv * j + t, i * (2 * rb) + rb + iv * j + t)
            return c

        wait_batch(sem0)
        pltpu.async_copy(keys_hbm.at[pl.ds(r0 + rb, rb)], buf1, sem1)
        lax.fori_loop(0, rb // iv, inner0, carry)
        wait_batch(sem1)
        nxt = jnp.minimum(r0 + 2 * rb, _CN - rb)
        pltpu.async_copy(keys_hbm.at[pl.ds(nxt, rb)], buf0, sem0)
        lax.fori_loop(0, rb // iv, inner1, carry)
        return carry

    lax.fori_loop(0, _RPW // (2 * rb), outer, jnp.int32(0))
    wait_batch(sem0)
    ow = _RPW * _KP1
    pltpu.sync_copy(ob.at[pl.ds(0, ow)], out_hbm.at[pl.ds(wid * ow, ow)])


def _sc_topk(keys):
    mesh = plsc.VectorSubcoreMesh(core_axis_name="c", subcore_axis_name="s",
                                  num_cores=_NC, num_subcores=_NS)
    f = pl.kernel(
        _sc_topk_body,
        out_type=jax.ShapeDtypeStruct((_CN * _KP1,), jnp.float32),
        mesh=mesh,
        compiler_params=pltpu.CompilerParams(needs_layout_passes=False),
        scratch_types=[
            pltpu.VMEM((8, _N), jnp.int32),
            pltpu.VMEM((8, _N), jnp.int32),
            pltpu.VMEM((_RPW * _KP1 + 16,), jnp.float32),
            pltpu.SemaphoreType.DMA,
            pltpu.SemaphoreType.DMA,
        ],
    )
    return f(keys)


def kernel(positions, k):
    del k
    outs = []
    for c in range(_NCH):
        keys = _tc_pack(positions, c)
        outs.append(_sc_topk(keys))
    return jnp.concatenate(outs).reshape(_N, _KP1)


# speedup vs baseline: 1.3359x; 1.3359x over previous
import functools

import jax
import jax.numpy as jnp
from jax import lax
from jax.experimental import pallas as pl
from jax.experimental.pallas import tpu as pltpu
from jax.experimental.pallas import tpu_sc as plsc

_N = 2048
_D = 64
_KP1 = 17
_BLK = 256
_KEY_MASK = ~2047
_BIG = 0x7FFFFFFF

_NC = 2
_NS = 16
_NW = _NC * _NS
_NCH = 1
_CN = _N // _NCH
_RPW = _CN // _NW


def _tc_pack_body(chunk, pos_ref, out_ref):
    i = pl.program_id(0)
    p = pos_ref[...]
    a = pos_ref[pl.ds(chunk * _CN + i * _BLK, _BLK), :]
    g = lax.dot_general(a, p, (((1,), (1,)), ((), ())),
                        preferred_element_type=jnp.float32)
    na = jnp.sum(a * a, axis=1, keepdims=True)
    nb = jnp.sum(p * p, axis=1)[None, :]
    s = jnp.maximum(na + nb - 2.0 * g, 0.0)
    col = lax.broadcasted_iota(jnp.int32, s.shape, 1)
    row = lax.broadcasted_iota(jnp.int32, s.shape, 0) + chunk * _CN + i * _BLK
    s = jnp.where(col == row, 0.0, s)
    nrm = jnp.sqrt(s)
    bits = lax.bitcast_convert_type(nrm, jnp.int32)
    out_ref[...] = (bits & jnp.int32(_KEY_MASK)) | col


def _tc_pack(positions, chunk):
    return pl.pallas_call(
        functools.partial(_tc_pack_body, chunk),
        grid=(_CN // _BLK,),
        in_specs=[
            pl.BlockSpec((_N, _D), lambda i: (0, 0)),
        ],
        out_specs=pl.BlockSpec((_BLK, _N), lambda i: (i, 0)),
        out_shape=jax.ShapeDtypeStruct((_CN, _N), jnp.int32),
    )(positions)


def _tree_min(vs):
    while len(vs) > 1:
        nxt = [jnp.minimum(vs[i], vs[i + 1]) for i in range(0, len(vs) - 1, 2)]
        if len(vs) % 2:
            nxt.append(vs[-1])
        vs = nxt
    return vs[0]


def _merge2(ra, da, rb, db):
    cd = lax.rev(rb, (0,))
    m = jnp.minimum(ra, cd)
    x = jnp.maximum(ra, cd)
    return jnp.sort(m), jnp.minimum(jnp.minimum(da, db), jnp.min(x))


def _merge_tree(sorted_chunks):
    big = jnp.int32(_BIG)
    lvl = [(c, big) for c in sorted_chunks]
    while len(lvl) > 1:
        nxt = [_merge2(*lvl[i], *lvl[i + 1]) for i in range(0, len(lvl) - 1, 2)]
        if len(lvl) % 2:
            nxt.append(lvl[-1])
        lvl = nxt
    return lvl[0]


def _sc_topk_body(keys_hbm, out_hbm, buf0, buf1, ob, sem0, sem1):
    wid = lax.axis_index("s") * _NC + lax.axis_index("c")
    base_row = wid * _RPW
    gather_off = lax.iota(jnp.int32, 16) * 16
    big16 = jnp.full((16,), _BIG, jnp.int32)

    def row_topk(buf, row, slot):
        groups = []
        for g in range(8):
            chunks = [buf[row, pl.ds(g * 256 + i * 16, 16)] for i in range(16)]
            groups.append(_tree_min(chunks))
        r, d = _merge_tree([jnp.sort(g) for g in groups])
        row16 = jnp.broadcast_to(row, (16,))
        def block_chunk(key_scalar):
            c = key_scalar & 2047
            base = lax.shift_right_logical(c, 8) * 256 + (c & 15)
            return plsc.load_gather(buf, [row16, gather_off + base])

        cands = [jnp.sort(block_chunk(r[j])) for j in range(16)]
        cands.append(jnp.sort(block_chunk(d)))
        r2, d2 = _merge_tree(cands)
        km = jnp.int32(_KEY_MASK)
        ob[pl.ds(slot * _KP1 + 16, 16)] = plsc.bitcast(
            jnp.broadcast_to(d2 & km, (16,)), jnp.float32)
        ob[pl.ds(slot * _KP1, 16)] = plsc.bitcast(r2 & km, jnp.float32)

    rb = 8

    def wait_batch(sem):
        pltpu.make_async_copy(keys_hbm.at[pl.ds(0, rb)], buf0, sem).wait()

    pltpu.async_copy(keys_hbm.at[pl.ds(base_row, rb)], buf0, sem0)

    def outer(i, carry):
        r0 = base_row + i * (2 * rb)

        iv = 2

        def inner0(j, c):
            for t in range(iv):
                row_topk(buf0, iv * j + t, i * (2 * rb) + iv * j + t)
            return c

        def inner1(j, c):
            for t in range(iv):
                row_topk(buf1, iv * j + t, i * (2 * rb) + rb + iv * j + t)
            return c

        wait_batch(sem0)
        pltpu.async_copy(keys_hbm.at[pl.ds(r0 + rb, rb)], buf1, sem1)
        lax.fori_loop(0, rb // iv, inner0, carry)
        wait_batch(sem1)
        nxt = jnp.minimum(r0 + 2 * rb, _CN - rb)
        pltpu.async_copy(keys_hbm.at[pl.ds(nxt, rb)], buf0, sem0)
        lax.fori_loop(0, rb // iv, inner1, carry)
        return carry

    lax.fori_loop(0, _RPW // (2 * rb), outer, jnp.int32(0))
    wait_batch(sem0)
    ow = _RPW * _KP1
    pltpu.sync_copy(ob.at[pl.ds(0, ow)], out_hbm.at[pl.ds(wid * ow, ow)])


def _sc_topk(keys):
    mesh = plsc.VectorSubcoreMesh(core_axis_name="c", subcore_axis_name="s",
                                  num_cores=_NC, num_subcores=_NS)
    f = pl.kernel(
        _sc_topk_body,
        out_type=jax.ShapeDtypeStruct((_CN * _KP1,), jnp.float32),
        mesh=mesh,
        compiler_params=pltpu.CompilerParams(needs_layout_passes=False),
        scratch_types=[
            pltpu.VMEM((8, _N), jnp.int32),
            pltpu.VMEM((8, _N), jnp.int32),
            pltpu.VMEM((_RPW * _KP1 + 16,), jnp.float32),
            pltpu.SemaphoreType.DMA,
            pltpu.SemaphoreType.DMA,
        ],
    )
    return f(keys)


def kernel(positions, k):
    del k
    outs = []
    for c in range(_NCH):
        keys = _tc_pack(positions, c)
        outs.append(_sc_topk(keys))
    return jnp.concatenate(outs).reshape(_N, _KP1)


# speedup vs baseline: 1.4676x; 1.0985x over previous
import functools

import jax
import jax.numpy as jnp
from jax import lax
from jax.experimental import pallas as pl
from jax.experimental.pallas import tpu as pltpu
from jax.experimental.pallas import tpu_sc as plsc

_N = 2048
_D = 64
_KP1 = 17
_BLK = 256
_KEY_MASK = ~2047
_BIG = 0x7FFFFFFF

_NC = 2
_NS = 16
_NW = _NC * _NS
_NCH = 1
_CN = _N // _NCH
_RPW = _CN // _NW


def _tc_pack_body(chunk, pos_ref, out_ref):
    i = pl.program_id(0)
    p = pos_ref[...]
    a = pos_ref[pl.ds(chunk * _CN + i * _BLK, _BLK), :]
    g = lax.dot_general(a, p, (((1,), (1,)), ((), ())),
                        preferred_element_type=jnp.float32)
    na = jnp.sum(a * a, axis=1, keepdims=True)
    nb = jnp.sum(p * p, axis=1)[None, :]
    s = jnp.maximum(na + nb - 2.0 * g, 0.0)
    col = lax.broadcasted_iota(jnp.int32, s.shape, 1)
    row = lax.broadcasted_iota(jnp.int32, s.shape, 0) + chunk * _CN + i * _BLK
    s = jnp.where(col == row, 0.0, s)
    nrm = jnp.sqrt(s)
    bits = lax.bitcast_convert_type(nrm, jnp.int32)
    out_ref[...] = (bits & jnp.int32(_KEY_MASK)) | col


def _tc_pack(positions, chunk):
    return pl.pallas_call(
        functools.partial(_tc_pack_body, chunk),
        grid=(_CN // _BLK,),
        in_specs=[
            pl.BlockSpec((_N, _D), lambda i: (0, 0)),
        ],
        out_specs=pl.BlockSpec((_BLK, _N), lambda i: (i, 0)),
        out_shape=jax.ShapeDtypeStruct((_CN, _N), jnp.int32),
    )(positions)


def _tree_min(vs):
    while len(vs) > 1:
        nxt = [jnp.minimum(vs[i], vs[i + 1]) for i in range(0, len(vs) - 1, 2)]
        if len(vs) % 2:
            nxt.append(vs[-1])
        vs = nxt
    return vs[0]


def _merge2(ra, da, rb, db):
    cd = lax.rev(rb, (0,))
    m = jnp.minimum(ra, cd)
    x = jnp.maximum(ra, cd)
    return jnp.sort(m), jnp.minimum(jnp.minimum(da, db), jnp.min(x))


def _merge_tree(sorted_chunks):
    big = jnp.int32(_BIG)
    lvl = [(c, big) for c in sorted_chunks]
    while len(lvl) > 1:
        nxt = [_merge2(*lvl[i], *lvl[i + 1]) for i in range(0, len(lvl) - 1, 2)]
        if len(lvl) % 2:
            nxt.append(lvl[-1])
        lvl = nxt
    return lvl[0]


def _sc_topk_body(keys_hbm, out_hbm, buf0, buf1, ob, sem0, sem1):
    wid = lax.axis_index("s") * _NC + lax.axis_index("c")
    base_row = wid * _RPW
    gather_off = lax.iota(jnp.int32, 16) * 16
    big16 = jnp.full((16,), _BIG, jnp.int32)

    def row_topk(buf, row, slot):
        groups = []
        for g in range(8):
            chunks = [buf[row, pl.ds(g * 256 + i * 16, 16)] for i in range(16)]
            groups.append(_tree_min(chunks))
        r, d = _merge_tree([jnp.sort(g) for g in groups])
        row16 = jnp.broadcast_to(row, (16,))
        base16 = r & jnp.int32(0x70F)
        cands = [jnp.sort(plsc.load_gather(buf, [row16, base16 + 16 * k]))
                 for k in range(16)]
        bd = d & jnp.int32(0x70F)
        cands.append(jnp.sort(plsc.load_gather(buf, [row16, gather_off + bd])))
        r2, d2 = _merge_tree(cands)
        km = jnp.int32(_KEY_MASK)
        ob[pl.ds(slot * _KP1 + 16, 16)] = plsc.bitcast(
            jnp.broadcast_to(d2 & km, (16,)), jnp.float32)
        ob[pl.ds(slot * _KP1, 16)] = plsc.bitcast(r2 & km, jnp.float32)

    rb = 8

    def wait_batch(sem):
        pltpu.make_async_copy(keys_hbm.at[pl.ds(0, rb)], buf0, sem).wait()

    pltpu.async_copy(keys_hbm.at[pl.ds(base_row, rb)], buf0, sem0)

    def outer(i, carry):
        r0 = base_row + i * (2 * rb)

        iv = 2

        def inner0(j, c):
            for t in range(iv):
                row_topk(buf0, iv * j + t, i * (2 * rb) + iv * j + t)
            return c

        def inner1(j, c):
            for t in range(iv):
                row_topk(buf1, iv * j + t, i * (2 * rb) + rb + iv * j + t)
            return c

        wait_batch(sem0)
        pltpu.async_copy(keys_hbm.at[pl.ds(r0 + rb, rb)], buf1, sem1)
        lax.fori_loop(0, rb // iv, inner0, carry)
        wait_batch(sem1)
        nxt = jnp.minimum(r0 + 2 * rb, _CN - rb)
        pltpu.async_copy(keys_hbm.at[pl.ds(nxt, rb)], buf0, sem0)
        lax.fori_loop(0, rb // iv, inner1, carry)
        return carry

    lax.fori_loop(0, _RPW // (2 * rb), outer, jnp.int32(0))
    wait_batch(sem0)
    ow = _RPW * _KP1
    pltpu.sync_copy(ob.at[pl.ds(0, ow)], out_hbm.at[pl.ds(wid * ow, ow)])


def _sc_topk(keys):
    mesh = plsc.VectorSubcoreMesh(core_axis_name="c", subcore_axis_name="s",
                                  num_cores=_NC, num_subcores=_NS)
    f = pl.kernel(
        _sc_topk_body,
        out_type=jax.ShapeDtypeStruct((_CN * _KP1,), jnp.float32),
        mesh=mesh,
        compiler_params=pltpu.CompilerParams(needs_layout_passes=False),
        scratch_types=[
            pltpu.VMEM((8, _N), jnp.int32),
            pltpu.VMEM((8, _N), jnp.int32),
            pltpu.VMEM((_RPW * _KP1 + 16,), jnp.float32),
            pltpu.SemaphoreType.DMA,
            pltpu.SemaphoreType.DMA,
        ],
    )
    return f(keys)


def kernel(positions, k):
    del k
    outs = []
    for c in range(_NCH):
        keys = _tc_pack(positions, c)
        outs.append(_sc_topk(keys))
    return jnp.concatenate(outs).reshape(_N, _KP1)


# speedup vs baseline: 1.5502x; 1.0563x over previous
import functools

import jax
import jax.numpy as jnp
from jax import lax
from jax.experimental import pallas as pl
from jax.experimental.pallas import tpu as pltpu
from jax.experimental.pallas import tpu_sc as plsc

_N = 2048
_D = 64
_KP1 = 17
_BLK = 256
_KEY_MASK = ~2047
_BIG = 0x7FFFFFFF

_NC = 2
_NS = 16
_NW = _NC * _NS
_NCH = 1
_CN = _N // _NCH
_RPW = _CN // _NW


def _tc_pack_body(chunk, pos_ref, out_ref):
    i = pl.program_id(0)
    p = pos_ref[...]
    a = pos_ref[pl.ds(chunk * _CN + i * _BLK, _BLK), :]
    g = lax.dot_general(a, p, (((1,), (1,)), ((), ())),
                        preferred_element_type=jnp.float32)
    na = jnp.sum(a * a, axis=1, keepdims=True)
    nb = jnp.sum(p * p, axis=1)[None, :]
    s = jnp.maximum(na + nb - 2.0 * g, 0.0)
    col = lax.broadcasted_iota(jnp.int32, s.shape, 1)
    row = lax.broadcasted_iota(jnp.int32, s.shape, 0) + chunk * _CN + i * _BLK
    s = jnp.where(col == row, 0.0, s)
    nrm = jnp.sqrt(s)
    bits = lax.bitcast_convert_type(nrm, jnp.int32)
    k16 = lax.shift_right_logical(bits, 16)
    lo = k16[:, : _N // 2]
    hi = k16[:, _N // 2:]
    out_ref[...] = lo | (hi << 16)


def _tc_pack(positions, chunk):
    return pl.pallas_call(
        functools.partial(_tc_pack_body, chunk),
        grid=(_CN // _BLK,),
        in_specs=[
            pl.BlockSpec((_N, _D), lambda i: (0, 0)),
        ],
        out_specs=pl.BlockSpec((_BLK, _N // 2), lambda i: (i, 0)),
        out_shape=jax.ShapeDtypeStruct((_CN, _N // 2), jnp.int32),
    )(positions)


def _tree_min(vs):
    while len(vs) > 1:
        nxt = [jnp.minimum(vs[i], vs[i + 1]) for i in range(0, len(vs) - 1, 2)]
        if len(vs) % 2:
            nxt.append(vs[-1])
        vs = nxt
    return vs[0]


def _merge2(ra, da, rb, db):
    cd = lax.rev(rb, (0,))
    m = jnp.minimum(ra, cd)
    x = jnp.maximum(ra, cd)
    return jnp.sort(m), jnp.minimum(jnp.minimum(da, db), jnp.min(x))


def _merge_tree(sorted_chunks):
    big = jnp.int32(_BIG)
    lvl = [(c, big) for c in sorted_chunks]
    while len(lvl) > 1:
        nxt = [_merge2(*lvl[i], *lvl[i + 1]) for i in range(0, len(lvl) - 1, 2)]
        if len(lvl) % 2:
            nxt.append(lvl[-1])
        lvl = nxt
    return lvl[0]


def _sc_topk_body(keys_hbm, out_hbm, buf0, buf1, ob, sem0, sem1):
    wid = lax.axis_index("s") * _NC + lax.axis_index("c")
    base_row = wid * _RPW
    iota16 = lax.iota(jnp.int32, 16)
    two_iota = iota16 * 2
    iota8m = iota16 & 7
    mask8 = iota16 < 8
    c16 = jnp.int32(0xFFFF)

    def row_topk(buf, row, slot):
        gms = []
        for g in range(8):
            chunks = [
                plsc.bitcast(buf[row, pl.ds(g * 128 + i * 16, 16)], jnp.int16)
                for i in range(8)]
            gms.append(_tree_min(chunks))
        leaves = []
        for g, gm in enumerate(gms):
            x32 = plsc.bitcast(gm, jnp.int32)
            e = x32 & c16
            o = lax.shift_right_logical(x32, 16)
            leaves.append((e << 8) | (two_iota + g * 32))
            leaves.append((o << 8) | (two_iota + g * 32 + 1))
        r, d = _merge_tree([jnp.sort(v) for v in leaves])
        row16 = jnp.broadcast_to(row, (16,))
        ids = r & 255
        wb = lax.shift_right_logical(ids, 5) * 128 + \
            lax.shift_right_logical(ids & 31, 1)
        h = ids & 1
        cands = []
        for k in range(8):
            w = plsc.load_gather(buf, [row16, wb + 16 * k])
            cands.append(jnp.where(h == 1,
                                   lax.shift_right_logical(w, 16), w & c16))
        idd = d & 255
        wbd = lax.shift_right_logical(idd, 5) * 128 + \
            lax.shift_right_logical(idd & 31, 1)
        hd = (idd & 1) * 16
        w = plsc.load_gather(buf, [row16, wbd + 16 * iota8m])
        v17 = lax.shift_right_logical(w, hd) & c16
        cands.append(jnp.where(mask8, v17, jnp.int32(_BIG)))
        r2, d2 = _merge_tree([jnp.sort(v) for v in cands])
        ob[pl.ds(slot * _KP1 + 16, 16)] = plsc.bitcast(
            jnp.broadcast_to(d2 << 16, (16,)), jnp.float32)
        ob[pl.ds(slot * _KP1, 16)] = plsc.bitcast(r2 << 16, jnp.float32)

    rb = 8

    def wait_batch(sem):
        pltpu.make_async_copy(keys_hbm.at[pl.ds(0, rb)], buf0, sem).wait()

    pltpu.async_copy(keys_hbm.at[pl.ds(base_row, rb)], buf0, sem0)

    def outer(i, carry):
        r0 = base_row + i * (2 * rb)

        iv = 2

        def inner0(j, c):
            for t in range(iv):
                row_topk(buf0, iv * j + t, i * (2 * rb) + iv * j + t)
            return c

        def inner1(j, c):
            for t in range(iv):
                row_topk(buf1, iv * j + t, i * (2 * rb) + rb + iv * j + t)
            return c

        wait_batch(sem0)
        pltpu.async_copy(keys_hbm.at[pl.ds(r0 + rb, rb)], buf1, sem1)
        lax.fori_loop(0, rb // iv, inner0, carry)
        wait_batch(sem1)
        nxt = jnp.minimum(r0 + 2 * rb, _CN - rb)
        pltpu.async_copy(keys_hbm.at[pl.ds(nxt, rb)], buf0, sem0)
        lax.fori_loop(0, rb // iv, inner1, carry)
        return carry

    lax.fori_loop(0, _RPW // (2 * rb), outer, jnp.int32(0))
    wait_batch(sem0)
    ow = _RPW * _KP1
    pltpu.sync_copy(ob.at[pl.ds(0, ow)], out_hbm.at[pl.ds(wid * ow, ow)])


def _sc_topk(keys):
    mesh = plsc.VectorSubcoreMesh(core_axis_name="c", subcore_axis_name="s",
                                  num_cores=_NC, num_subcores=_NS)
    f = pl.kernel(
        _sc_topk_body,
        out_type=jax.ShapeDtypeStruct((_CN * _KP1,), jnp.float32),
        mesh=mesh,
        compiler_params=pltpu.CompilerParams(needs_layout_passes=False),
        scratch_types=[
            pltpu.VMEM((8, _N // 2), jnp.int32),
            pltpu.VMEM((8, _N // 2), jnp.int32),
            pltpu.VMEM((_RPW * _KP1 + 16,), jnp.float32),
            pltpu.SemaphoreType.DMA,
            pltpu.SemaphoreType.DMA,
        ],
    )
    return f(keys)


def kernel(positions, k):
    del k
    outs = []
    for c in range(_NCH):
        keys = _tc_pack(positions, c)
        outs.append(_sc_topk(keys))
    return jnp.concatenate(outs).reshape(_N, _KP1)


# speedup vs baseline: 1.6370x; 1.0560x over previous
import functools

import jax
import jax.numpy as jnp
from jax import lax
from jax.experimental import pallas as pl
from jax.experimental.pallas import tpu as pltpu
from jax.experimental.pallas import tpu_sc as plsc

_N = 2048
_D = 64
_KP1 = 17
_BLK = 256
_KEY_MASK = ~2047
_BIG = 0x7FFFFFFF

_NC = 2
_NS = 16
_NW = _NC * _NS
_NCH = 1
_CN = _N // _NCH
_RPW = _CN // _NW


def _tc_pack_body(chunk, pos_hbm, out_ref, pos_vmem, sem):
    i = pl.program_id(0)

    @pl.when(i == 0)
    def _stage_positions():
        cp = pltpu.make_async_copy(pos_hbm, pos_vmem, sem)
        cp.start()
        cp.wait()

    p = pos_vmem[...]
    a = pos_vmem[pl.ds(chunk * _CN + i * _BLK, _BLK), :]
    g = lax.dot_general(a, p, (((1,), (1,)), ((), ())),
                        preferred_element_type=jnp.float32)
    na = jnp.sum(a * a, axis=1, keepdims=True)
    nb = jnp.sum(p * p, axis=1)[None, :]
    s = jnp.maximum(na + nb - 2.0 * g, 0.0)
    col = lax.broadcasted_iota(jnp.int32, s.shape, 1)
    row = lax.broadcasted_iota(jnp.int32, s.shape, 0) + chunk * _CN + i * _BLK
    s = jnp.where(col == row, 0.0, s)
    nrm = s * lax.rsqrt(s + 1e-30)
    bits = lax.bitcast_convert_type(nrm, jnp.int32)
    k16 = lax.shift_right_logical(bits, 16)
    lo = k16[:, : _N // 2]
    hi = k16[:, _N // 2:]
    out_ref[...] = lo | (hi << 16)


def _tc_pack(positions, chunk):
    return pl.pallas_call(
        functools.partial(_tc_pack_body, chunk),
        grid=(_CN // _BLK,),
        in_specs=[
            pl.BlockSpec(memory_space=pl.ANY),
        ],
        out_specs=pl.BlockSpec((_BLK, _N // 2), lambda i: (i, 0)),
        out_shape=jax.ShapeDtypeStruct((_CN, _N // 2), jnp.int32),
        scratch_shapes=[
            pltpu.VMEM((_N, _D), jnp.float32),
            pltpu.SemaphoreType.DMA,
        ],
    )(positions)


def _tree_min(vs):
    while len(vs) > 1:
        nxt = [jnp.minimum(vs[i], vs[i + 1]) for i in range(0, len(vs) - 1, 2)]
        if len(vs) % 2:
            nxt.append(vs[-1])
        vs = nxt
    return vs[0]


def _merge2(ra, da, rb, db):
    cd = lax.rev(rb, (0,))
    m = jnp.minimum(ra, cd)
    x = jnp.maximum(ra, cd)
    return jnp.sort(m), jnp.minimum(jnp.minimum(da, db), jnp.min(x))


def _merge_tree(sorted_chunks):
    big = jnp.int32(_BIG)
    lvl = [(c, big) for c in sorted_chunks]
    while len(lvl) > 1:
        nxt = [_merge2(*lvl[i], *lvl[i + 1]) for i in range(0, len(lvl) - 1, 2)]
        if len(lvl) % 2:
            nxt.append(lvl[-1])
        lvl = nxt
    return lvl[0]


def _sc_topk_body(keys_hbm, out_hbm, buf0, buf1, ob, sem0, sem1):
    wid = lax.axis_index("s") * _NC + lax.axis_index("c")
    base_row = wid * _RPW
    iota16 = lax.iota(jnp.int32, 16)
    two_iota = iota16 * 2
    iota8m = iota16 & 7
    mask8 = iota16 < 8
    c16 = jnp.int32(0xFFFF)

    def row_topk(buf, row, slot):
        gms = []
        for g in range(8):
            chunks = [
                plsc.bitcast(buf[row, pl.ds(g * 128 + i * 16, 16)], jnp.int16)
                for i in range(8)]
            gms.append(_tree_min(chunks))
        leaves = []
        for g, gm in enumerate(gms):
            x32 = plsc.bitcast(gm, jnp.int32)
            e = x32 & c16
            o = lax.shift_right_logical(x32, 16)
            leaves.append((e << 8) | (two_iota + g * 32))
            leaves.append((o << 8) | (two_iota + g * 32 + 1))
        r, d = _merge_tree([jnp.sort(v) for v in leaves])
        row16 = jnp.broadcast_to(row, (16,))
        ids = r & 255
        wb = lax.shift_right_logical(ids, 5) * 128 + \
            lax.shift_right_logical(ids & 31, 1)
        h = ids & 1
        cands = []
        for k in range(8):
            w = plsc.load_gather(buf, [row16, wb + 16 * k])
            cands.append(jnp.where(h == 1,
                                   lax.shift_right_logical(w, 16), w & c16))
        idd = d & 255
        wbd = lax.shift_right_logical(idd, 5) * 128 + \
            lax.shift_right_logical(idd & 31, 1)
        hd = (idd & 1) * 16
        w = plsc.load_gather(buf, [row16, wbd + 16 * iota8m])
        v17 = lax.shift_right_logical(w, hd) & c16
        cands.append(jnp.where(mask8, v17, jnp.int32(_BIG)))
        r2, d2 = _merge_tree([jnp.sort(v) for v in cands])
        ob[pl.ds(slot * _KP1 + 16, 16)] = plsc.bitcast(
            jnp.broadcast_to(d2 << 16, (16,)), jnp.float32)
        ob[pl.ds(slot * _KP1, 16)] = plsc.bitcast(r2 << 16, jnp.float32)

    rb = 8

    def wait_batch(sem):
        pltpu.make_async_copy(keys_hbm.at[pl.ds(0, rb)], buf0, sem).wait()

    pltpu.async_copy(keys_hbm.at[pl.ds(base_row, rb)], buf0, sem0)

    def outer(i, carry):
        r0 = base_row + i * (2 * rb)

        iv = 2

        def inner0(j, c):
            for t in range(iv):
                row_topk(buf0, iv * j + t, i * (2 * rb) + iv * j + t)
            return c

        def inner1(j, c):
            for t in range(iv):
                row_topk(buf1, iv * j + t, i * (2 * rb) + rb + iv * j + t)
            return c

        wait_batch(sem0)
        pltpu.async_copy(keys_hbm.at[pl.ds(r0 + rb, rb)], buf1, sem1)
        lax.fori_loop(0, rb // iv, inner0, carry)
        wait_batch(sem1)
        nxt = jnp.minimum(r0 + 2 * rb, _CN - rb)
        pltpu.async_copy(keys_hbm.at[pl.ds(nxt, rb)], buf0, sem0)
        lax.fori_loop(0, rb // iv, inner1, carry)
        return carry

    lax.fori_loop(0, _RPW // (2 * rb), outer, jnp.int32(0))
    wait_batch(sem0)
    ow = _RPW * _KP1
    pltpu.sync_copy(ob.at[pl.ds(0, ow)], out_hbm.at[pl.ds(wid * ow, ow)])


def _sc_topk(keys):
    mesh = plsc.VectorSubcoreMesh(core_axis_name="c", subcore_axis_name="s",
                                  num_cores=_NC, num_subcores=_NS)
    f = pl.kernel(
        _sc_topk_body,
        out_type=jax.ShapeDtypeStruct((_CN * _KP1,), jnp.float32),
        mesh=mesh,
        compiler_params=pltpu.CompilerParams(needs_layout_passes=False),
        scratch_types=[
            pltpu.VMEM((8, _N // 2), jnp.int32),
            pltpu.VMEM((8, _N // 2), jnp.int32),
            pltpu.VMEM((_RPW * _KP1 + 16,), jnp.float32),
            pltpu.SemaphoreType.DMA,
            pltpu.SemaphoreType.DMA,
        ],
    )
    return f(keys)


def kernel(positions, k):
    del k
    outs = []
    for c in range(_NCH):
        keys = _tc_pack(positions, c)
        outs.append(_sc_topk(keys))
    return jnp.concatenate(outs).reshape(_N, _KP1)


# speedup vs baseline: 1.6893x; 1.0320x over previous
import functools

import jax
import jax.numpy as jnp
from jax import lax
from jax.experimental import pallas as pl
from jax.experimental.pallas import tpu as pltpu
from jax.experimental.pallas import tpu_sc as plsc

_N = 2048
_D = 64
_KP1 = 17
_BLK = 512
_KEY_MASK = ~2047
_BIG = 0x7FFFFFFF

_NC = 2
_NS = 16
_NW = _NC * _NS
_NCH = 1
_CN = _N // _NCH
_RPW = _CN // _NW


def _tc_pack_body(chunk, pos_hbm, out_ref, pos_vmem, sem):
    i = pl.program_id(0)

    @pl.when(i == 0)
    def _stage_positions():
        cp = pltpu.make_async_copy(pos_hbm, pos_vmem, sem)
        cp.start()
        cp.wait()

    p = pos_vmem[...]
    a = pos_vmem[pl.ds(chunk * _CN + i * _BLK, _BLK), :]
    g = lax.dot_general(a, p, (((1,), (1,)), ((), ())),
                        preferred_element_type=jnp.float32)
    na = jnp.sum(a * a, axis=1, keepdims=True)
    nb = jnp.sum(p * p, axis=1)[None, :]
    s = jnp.maximum(na + nb - 2.0 * g, 0.0)
    col = lax.broadcasted_iota(jnp.int32, s.shape, 1)
    row = lax.broadcasted_iota(jnp.int32, s.shape, 0) + chunk * _CN + i * _BLK
    s = jnp.where(col == row, 0.0, s)
    nrm = s * lax.rsqrt(s + 1e-30)
    bits = lax.bitcast_convert_type(nrm, jnp.int32)
    k16 = lax.shift_right_logical(bits, 16)
    lo = k16[:, : _N // 2]
    hi = k16[:, _N // 2:]
    out_ref[...] = lo | (hi << 16)


def _tc_pack(positions, chunk):
    return pl.pallas_call(
        functools.partial(_tc_pack_body, chunk),
        grid=(_CN // _BLK,),
        in_specs=[
            pl.BlockSpec(memory_space=pl.ANY),
        ],
        out_specs=pl.BlockSpec((_BLK, _N // 2), lambda i: (i, 0)),
        out_shape=jax.ShapeDtypeStruct((_CN, _N // 2), jnp.int32),
        scratch_shapes=[
            pltpu.VMEM((_N, _D), jnp.float32),
            pltpu.SemaphoreType.DMA,
        ],
    )(positions)


def _tree_min(vs):
    while len(vs) > 1:
        nxt = [jnp.minimum(vs[i], vs[i + 1]) for i in range(0, len(vs) - 1, 2)]
        if len(vs) % 2:
            nxt.append(vs[-1])
        vs = nxt
    return vs[0]


def _merge2(ra, da, rb, db):
    cd = lax.rev(rb, (0,))
    m = jnp.minimum(ra, cd)
    x = jnp.maximum(ra, cd)
    return jnp.sort(m), jnp.minimum(jnp.minimum(da, db), jnp.min(x))


def _merge_tree(sorted_chunks):
    big = jnp.int32(_BIG)
    lvl = [(c, big) for c in sorted_chunks]
    while len(lvl) > 1:
        nxt = [_merge2(*lvl[i], *lvl[i + 1]) for i in range(0, len(lvl) - 1, 2)]
        if len(lvl) % 2:
            nxt.append(lvl[-1])
        lvl = nxt
    return lvl[0]


def _sc_topk_body(keys_hbm, out_hbm, buf0, buf1, ob, sem0, sem1):
    wid = lax.axis_index("s") * _NC + lax.axis_index("c")
    base_row = wid * _RPW
    iota16 = lax.iota(jnp.int32, 16)
    two_iota = iota16 * 2
    iota8m = iota16 & 7
    mask8 = iota16 < 8
    c16 = jnp.int32(0xFFFF)

    def row_topk(buf, row, slot):
        gms = []
        for g in range(8):
            chunks = [
                plsc.bitcast(buf[row, pl.ds(g * 128 + i * 16, 16)], jnp.int16)
                for i in range(8)]
            gms.append(_tree_min(chunks))
        leaves = []
        for g, gm in enumerate(gms):
            x32 = plsc.bitcast(gm, jnp.int32)
            e = x32 & c16
            o = lax.shift_right_logical(x32, 16)
            leaves.append((e << 8) | (two_iota + g * 32))
            leaves.append((o << 8) | (two_iota + g * 32 + 1))
        r, d = _merge_tree([jnp.sort(v) for v in leaves])
        row16 = jnp.broadcast_to(row, (16,))
        ids = r & 255
        wb = lax.shift_right_logical(ids, 5) * 128 + \
            lax.shift_right_logical(ids & 31, 1)
        h = ids & 1
        cands = []
        for k in range(8):
            w = plsc.load_gather(buf, [row16, wb + 16 * k])
            cands.append(jnp.where(h == 1,
                                   lax.shift_right_logical(w, 16), w & c16))
        idd = d & 255
        wbd = lax.shift_right_logical(idd, 5) * 128 + \
            lax.shift_right_logical(idd & 31, 1)
        hd = (idd & 1) * 16
        w = plsc.load_gather(buf, [row16, wbd + 16 * iota8m])
        v17 = lax.shift_right_logical(w, hd) & c16
        cands.append(jnp.where(mask8, v17, jnp.int32(_BIG)))
        r2, d2 = _merge_tree([jnp.sort(v) for v in cands])
        ob[slot, pl.ds(1, 16)] = plsc.bitcast(
            jnp.broadcast_to(d2 << 16, (16,)), jnp.float32)
        ob[slot, pl.ds(0, 16)] = plsc.bitcast(r2 << 16, jnp.float32)

    rb = 8

    def wait_batch(sem):
        pltpu.make_async_copy(keys_hbm.at[pl.ds(0, rb)], buf0, sem).wait()

    pltpu.async_copy(keys_hbm.at[pl.ds(base_row, rb)], buf0, sem0)

    def outer(i, carry):
        r0 = base_row + i * (2 * rb)

        iv = 2

        def inner0(j, c):
            for t in range(iv):
                row_topk(buf0, iv * j + t, i * (2 * rb) + iv * j + t)
            return c

        def inner1(j, c):
            for t in range(iv):
                row_topk(buf1, iv * j + t, i * (2 * rb) + rb + iv * j + t)
            return c

        wait_batch(sem0)
        pltpu.async_copy(keys_hbm.at[pl.ds(r0 + rb, rb)], buf1, sem1)
        lax.fori_loop(0, rb // iv, inner0, carry)
        wait_batch(sem1)
        nxt = jnp.minimum(r0 + 2 * rb, _CN - rb)
        pltpu.async_copy(keys_hbm.at[pl.ds(nxt, rb)], buf0, sem0)
        lax.fori_loop(0, rb // iv, inner1, carry)
        return carry

    lax.fori_loop(0, _RPW // (2 * rb), outer, jnp.int32(0))
    wait_batch(sem0)
    pltpu.sync_copy(ob, out_hbm.at[pl.ds(base_row, _RPW), :])


def _sc_topk(keys):
    mesh = plsc.VectorSubcoreMesh(core_axis_name="c", subcore_axis_name="s",
                                  num_cores=_NC, num_subcores=_NS)
    f = pl.kernel(
        _sc_topk_body,
        out_type=jax.ShapeDtypeStruct((_CN, _KP1), jnp.float32),
        mesh=mesh,
        compiler_params=pltpu.CompilerParams(needs_layout_passes=False),
        scratch_types=[
            pltpu.VMEM((8, _N // 2), jnp.int32),
            pltpu.VMEM((8, _N // 2), jnp.int32),
            pltpu.VMEM((_RPW, _KP1), jnp.float32),
            pltpu.SemaphoreType.DMA,
            pltpu.SemaphoreType.DMA,
        ],
    )
    return f(keys)


def kernel(positions, k):
    del k
    outs = []
    for c in range(_NCH):
        keys = _tc_pack(positions, c)
        outs.append(_sc_topk(keys))
    return outs[0] if _NCH == 1 else jnp.concatenate(outs)
